# baseline (speedup 1.0000x reference)
import functools

import jax
import jax.numpy as jnp
from jax import lax
from jax.experimental import pallas as pl
from jax.experimental.pallas import tpu as pltpu
from jax.experimental.pallas import tpu_sc as plsc

EMB_DIM = 128
STRIPE = 128
HALVES = 2
SUB = STRIPE // HALVES
NBUF = 10
LA = 8


@functools.cache
def _build(batch: int, seq: int):
    info = plsc.get_sparse_core_info()
    nc, ns = info.num_cores, info.num_subcores
    nw = nc * ns
    assert batch == nw * STRIPE
    n_chunks = seq * HALVES
    assert LA < NBUF and n_chunks % NBUF == 0
    mesh = plsc.VectorSubcoreMesh(core_axis_name="c", subcore_axis_name="s")

    @functools.partial(
        pl.kernel,
        out_type=jax.ShapeDtypeStruct((seq, batch, EMB_DIM), jnp.float32),
        mesh=mesh,
        scratch_types=[
            pltpu.VMEM((seq, STRIPE), jnp.int32),
            pltpu.VMEM((NBUF, SUB, EMB_DIM), jnp.float32),
        ]
        + [pltpu.SemaphoreType.DMA] * (2 * NBUF),
    )
    def embed(idx_hbm, table_hbm, out_hbm, idx_v, rows_v, *sems):
        gs, ws = sems[:NBUF], sems[NBUF:]
        wid = lax.axis_index("s") * nc + lax.axis_index("c")
        col = wid * STRIPE
        pltpu.sync_copy(idx_hbm.at[:, pl.ds(col, STRIPE)], idx_v)

        def gather(c, b):
            s, off = c // HALVES, (c % HALVES) * SUB
            return pltpu.make_async_copy(
                table_hbm.at[idx_v.at[s, pl.ds(off, SUB)]], rows_v.at[b], gs[b])

        def write(c, b):
            s, off = c // HALVES, (c % HALVES) * SUB
            return pltpu.make_async_copy(
                rows_v.at[b], out_hbm.at[s, pl.ds(col + off, SUB)], ws[b])

        for c in range(LA):
            gather(c, c % NBUF).start()

        @pl.loop(0, n_chunks, step=NBUF)
        def _(i):
            for b in range(NBUF):
                c = i + b
                bf = (b + LA) % NBUF
                gather(c, b).wait()
                write(c, b).start()
                f = c + LA

                @pl.when(jnp.logical_and(f >= NBUF, f < n_chunks))
                def _():
                    write(f - NBUF, bf).wait()
                    gather(f, bf).start()

                if b + LA < NBUF:

                    @pl.when(f < NBUF)
                    def _():
                        gather(f, bf).start()

        for b in range(NBUF):
            write(n_chunks - NBUF + b, b).wait()

    return embed


def kernel(token_ids, weight):
    b, s = token_ids.shape
    embed = _build(b, s)
    out_sb = embed(token_ids.T.astype(jnp.int32), weight)
    return out_sb.transpose(1, 0, 2)
